# baseline (speedup 1.0000x reference)
import functools

import jax
import jax.numpy as jnp
from jax import lax
from jax.experimental import pallas as pl
from jax.experimental.pallas import tpu as pltpu
from jax.experimental.pallas import tpu_sc as plsc

_NC = 2
_NS = 16
_NW = _NC * _NS
_L = 16

_SG = 5
_BT = 128


@functools.lru_cache(maxsize=None)
def _emb_call(b: int, s: int, d: int):
    per_w_b = b // _NW
    per_w = per_w_b * s
    n_bt = per_w_b // _BT
    n_sg = s // _SG
    chunk = _SG * _BT
    n_chunks = n_bt * n_sg
    dt, dr = d // 8, 8

    mesh = plsc.VectorSubcoreMesh(core_axis_name="c", subcore_axis_name="s")

    @functools.partial(
        pl.kernel,
        mesh=mesh,
        out_type=jax.ShapeDtypeStruct((s, dt, b // _BT, dr, _BT), jnp.float32),
        compiler_params=pltpu.CompilerParams(
            use_tc_tiling_on_sc=False, needs_layout_passes=False),
        scratch_types=[
            pltpu.VMEM((per_w,), jnp.int32),
            pltpu.VMEM((chunk,), jnp.int32),
            pltpu.VMEM((chunk,), jnp.int32),
            pltpu.VMEM((chunk, d), jnp.float32),
            pltpu.VMEM((chunk, d), jnp.float32),
            pltpu.VMEM((_SG, dt, dr, _BT), jnp.float32),
            pltpu.VMEM((_SG, dt, dr, _BT), jnp.float32),
            pltpu.SemaphoreType.DMA,
            pltpu.SemaphoreType.DMA,
            pltpu.SemaphoreType.DMA,
            pltpu.SemaphoreType.DMA,
        ],
    )
    def k(x_hbm, table_hbm, out_hbm, idx_all, i0, i1, r0, r1, t0, t1,
          g0, g1, s0, s1):
        wid = lax.axis_index("s") * _NC + lax.axis_index("c")
        brow0 = wid * per_w_b
        bt0 = brow0 // _BT

        idx_c = (i0, i1)
        rows = (r0, r1)
        trans = (t0, t1)
        gsem = (g0, g1)
        ssem = (s0, s1)
        iota = lax.iota(jnp.int32, _L)

        pltpu.sync_copy(x_hbm.at[pl.ds(brow0 * s, per_w)], idx_all)

        def reorder_idx(c, slot):
            bti = c // n_sg
            sg = c % n_sg

            @plsc.parallel_loop(0, chunk // _L, unroll=8)
            def r_body(v):
                s_l = v // (_BT // _L)
                bv = v % (_BT // _L)
                src = (bti * _BT + bv * _L + iota) * s + (sg * _SG + s_l)
                vals = plsc.load_gather(idx_all, [src])
                idx_c[slot][pl.ds(v * _L, _L)] = vals

        def gather_start(slot):
            pltpu.async_copy(table_hbm.at[idx_c[slot]], rows[slot],
                             gsem[slot])

        def gather_wait(slot):
            pltpu.make_async_copy(table_hbm.at[idx_c[slot]], rows[slot],
                                  gsem[slot]).wait()

        def drain_stores(slot):
            h = pltpu.make_async_copy(
                trans[slot].at[0, 0], out_hbm.at[0, 0, 0], ssem[slot])
            for _ in range(_SG * dt):
                h.wait()

        def transpose_chunk(slot):
            rs = rows[slot]
            ts = trans[slot]

            @plsc.parallel_loop(0, _SG * d, unroll=8)
            def t_body(t2):
                s_l = t2 // d
                dd = t2 % d
                cid = jnp.full((_L,), dd, jnp.int32)
                for bv in range(_BT // _L):
                    rid = s_l * _BT + bv * _L + iota
                    vals = plsc.load_gather(rs, [rid, cid])
                    ts[s_l, dd // dr, dd % dr, pl.ds(bv * _L, _L)] = vals

        def store_chunk(c, slot):
            bti = c // n_sg
            sg = c % n_sg
            for s_l in range(_SG):
                for dti in range(dt):
                    pltpu.async_copy(
                        trans[slot].at[s_l, dti],
                        out_hbm.at[sg * _SG + s_l, dti, bt0 + bti],
                        ssem[slot])

        def run_chunk(c, slot):
            gather_wait(slot)

            @pl.when(c + 1 < n_chunks)
            def _():
                reorder_idx(c + 1, 1 - slot)
                gather_start(1 - slot)

            @pl.when(c >= 2)
            def _():
                drain_stores(slot)

            transpose_chunk(slot)
            store_chunk(c, slot)

        reorder_idx(0, 0)
        gather_start(0)

        def body(t, carry):
            run_chunk(2 * t, 0)
            run_chunk(2 * t + 1, 1)
            return carry

        lax.fori_loop(0, n_chunks // 2, body, 0)
        drain_stores(0)
        drain_stores(1)

    return k


def kernel(x, table):
    b, s = x.shape
    v, d = table.shape
    xf = jnp.minimum(x.reshape(b * s), v - 1)
    out5 = _emb_call(b, s, d)(xf, table)
    return out5.transpose(2, 4, 0, 1, 3).reshape(b, s, d)


# speedup vs baseline: 1.0063x; 1.0063x over previous
import functools

import jax
import jax.numpy as jnp
from jax import lax
from jax.experimental import pallas as pl
from jax.experimental.pallas import tpu as pltpu
from jax.experimental.pallas import tpu_sc as plsc

_NC = 2
_NS = 16
_NW = _NC * _NS
_L = 16

_SG = 5
_BT = 128


@functools.lru_cache(maxsize=None)
def _emb_call(b: int, s: int, d: int):
    per_w_b = b // _NW
    n_bt = per_w_b // _BT
    n_sg = s // _SG
    chunk = _SG * _BT
    n_chunks = n_bt * n_sg
    dt, dr = d // 8, 8

    mesh = plsc.VectorSubcoreMesh(core_axis_name="c", subcore_axis_name="s")

    @functools.partial(
        pl.kernel,
        mesh=mesh,
        out_type=jax.ShapeDtypeStruct((s, dt, b // _BT, dr, _BT), jnp.float32),
        compiler_params=pltpu.CompilerParams(
            use_tc_tiling_on_sc=False, needs_layout_passes=False),
        scratch_types=[
            pltpu.VMEM((chunk,), jnp.int32),
            pltpu.VMEM((chunk,), jnp.int32),
            pltpu.VMEM((chunk, d), jnp.float32),
            pltpu.VMEM((chunk, d), jnp.float32),
            pltpu.VMEM((_SG, dt, dr, _BT), jnp.float32),
            pltpu.VMEM((_SG, dt, dr, _BT), jnp.float32),
            pltpu.SemaphoreType.DMA,
            pltpu.SemaphoreType.DMA,
            pltpu.SemaphoreType.DMA,
            pltpu.SemaphoreType.DMA,
            pltpu.SemaphoreType.DMA,
            pltpu.SemaphoreType.DMA,
        ],
    )
    def k(xs_hbm, table_hbm, out_hbm, i0, i1, r0, r1, t0, t1,
          g0, g1, s0, s1, p0, p1):
        wid = lax.axis_index("s") * _NC + lax.axis_index("c")
        brow0 = wid * per_w_b
        bt0 = brow0 // _BT

        idx_c = (i0, i1)
        rows = (r0, r1)
        trans = (t0, t1)
        gsem = (g0, g1)
        ssem = (s0, s1)
        isem = (p0, p1)

        def idx_start(c, slot):
            bti = c // n_sg
            sg = c % n_sg
            for s_l in range(_SG):
                pltpu.async_copy(
                    xs_hbm.at[pl.ds((sg * _SG + s_l) * b + brow0 + bti * _BT,
                                    _BT)],
                    idx_c[slot].at[pl.ds(s_l * _BT, _BT)],
                    isem[slot])

        def idx_wait(slot):
            h = pltpu.make_async_copy(
                xs_hbm.at[pl.ds(0, _BT)],
                idx_c[slot].at[pl.ds(0, _BT)], isem[slot])
            for _ in range(_SG):
                h.wait()

        def gather_start(slot):
            pltpu.async_copy(table_hbm.at[idx_c[slot]], rows[slot],
                             gsem[slot])

        def gather_wait(slot):
            pltpu.make_async_copy(table_hbm.at[idx_c[slot]], rows[slot],
                                  gsem[slot]).wait()

        def drain_stores(slot):
            h = pltpu.make_async_copy(
                trans[slot].at[0, 0], out_hbm.at[0, 0, 0], ssem[slot])
            for _ in range(_SG * dt):
                h.wait()

        def transpose_chunk(slot):
            rs = rows[slot]
            ts = trans[slot]
            iota = lax.iota(jnp.int32, _L)

            @plsc.parallel_loop(0, _SG * d, unroll=8)
            def t_body(t2):
                s_l = t2 // d
                dd = t2 % d
                cid = jnp.full((_L,), dd, jnp.int32)
                for bv in range(_BT // _L):
                    rid = s_l * _BT + bv * _L + iota
                    vals = plsc.load_gather(rs, [rid, cid])
                    ts[s_l, dd // dr, dd % dr, pl.ds(bv * _L, _L)] = vals

        def store_chunk(c, slot):
            bti = c // n_sg
            sg = c % n_sg
            for s_l in range(_SG):
                for dti in range(dt):
                    pltpu.async_copy(
                        trans[slot].at[s_l, dti],
                        out_hbm.at[sg * _SG + s_l, dti, bt0 + bti],
                        ssem[slot])

        def run_chunk(c, slot):
            gather_wait(slot)

            @pl.when(c + 1 < n_chunks)
            def _():
                idx_wait(1 - slot)
                gather_start(1 - slot)

            @pl.when(c + 2 < n_chunks)
            def _():
                idx_start(c + 2, slot)

            @pl.when(c >= 2)
            def _():
                drain_stores(slot)

            transpose_chunk(slot)
            store_chunk(c, slot)

        idx_start(0, 0)
        idx_wait(0)
        gather_start(0)
        idx_start(1, 1)

        def body(t, carry):
            run_chunk(2 * t, 0)
            run_chunk(2 * t + 1, 1)
            return carry

        lax.fori_loop(0, n_chunks // 2, body, 0)
        drain_stores(0)
        drain_stores(1)

    return k


def kernel(x, table):
    b, s = x.shape
    v, d = table.shape
    xs = jnp.minimum(x.T.reshape(b * s), v - 1)
    out5 = _emb_call(b, s, d)(xs, table)
    return out5.transpose(2, 4, 0, 1, 3).reshape(b, s, d)


# speedup vs baseline: 1.3844x; 1.3758x over previous
import functools

import jax
import jax.numpy as jnp
from jax import lax
from jax.experimental import pallas as pl
from jax.experimental.pallas import tpu as pltpu
from jax.experimental.pallas import tpu_sc as plsc

_NC = 2
_NS = 16
_NW = _NC * _NS
_L = 16

_SG = 5
_BT = 128


@functools.lru_cache(maxsize=None)
def _emb_call(b: int, s: int, d: int):
    per_w_b = b // _NW
    n_bt = per_w_b // _BT
    n_sg = s // _SG
    chunk = _SG * _BT
    n_chunks = n_bt * n_sg
    dt, dr = d // 8, 8

    mesh = plsc.VectorSubcoreMesh(core_axis_name="c", subcore_axis_name="s")

    @functools.partial(
        pl.kernel,
        mesh=mesh,
        out_type=jax.ShapeDtypeStruct((s, dt, b // _BT, dr * _BT), jnp.float32),
        compiler_params=pltpu.CompilerParams(
            use_tc_tiling_on_sc=False, needs_layout_passes=False),
        scratch_types=[
            pltpu.VMEM((chunk,), jnp.int32),
            pltpu.VMEM((chunk,), jnp.int32),
            pltpu.VMEM((chunk, d), jnp.float32),
            pltpu.VMEM((chunk, d), jnp.float32),
            pltpu.VMEM((_SG, d * _BT), jnp.float32),
            pltpu.VMEM((_SG, d * _BT), jnp.float32),
            pltpu.SemaphoreType.DMA,
            pltpu.SemaphoreType.DMA,
            pltpu.SemaphoreType.DMA,
            pltpu.SemaphoreType.DMA,
            pltpu.SemaphoreType.DMA,
            pltpu.SemaphoreType.DMA,
        ],
    )
    def k(xs_hbm, table_hbm, out_hbm, i0, i1, r0, r1, t0, t1,
          g0, g1, s0, s1, p0, p1):
        wid = lax.axis_index("s") * _NC + lax.axis_index("c")
        brow0 = wid * per_w_b
        bt0 = brow0 // _BT

        idx_c = (i0, i1)
        rows = (r0, r1)
        trans = (t0, t1)
        gsem = (g0, g1)
        ssem = (s0, s1)
        isem = (p0, p1)

        def idx_start(c, slot):
            bti = c // n_sg
            sg = c % n_sg
            for s_l in range(_SG):
                pltpu.async_copy(
                    xs_hbm.at[pl.ds((sg * _SG + s_l) * b + brow0 + bti * _BT,
                                    _BT)],
                    idx_c[slot].at[pl.ds(s_l * _BT, _BT)],
                    isem[slot])

        def idx_wait(slot):
            h = pltpu.make_async_copy(
                xs_hbm.at[pl.ds(0, _BT)],
                idx_c[slot].at[pl.ds(0, _BT)], isem[slot])
            for _ in range(_SG):
                h.wait()

        def gather_start(slot):
            pltpu.async_copy(table_hbm.at[idx_c[slot]], rows[slot],
                             gsem[slot])

        def gather_wait(slot):
            pltpu.make_async_copy(table_hbm.at[idx_c[slot]], rows[slot],
                                  gsem[slot]).wait()

        def drain_stores(slot):
            h = pltpu.make_async_copy(
                trans[slot].at[0, pl.ds(0, dr * _BT)], out_hbm.at[0, 0, 0],
                ssem[slot])
            for _ in range(_SG * dt):
                h.wait()

        def transpose_chunk(slot):
            rs = rows[slot]
            ts = trans[slot]
            iota = lax.iota(jnp.int32, _L)
            n_tiles = _SG * (_BT // _L) * (d // _L)

            @plsc.parallel_loop(0, n_tiles, unroll=2)
            def t_body(t3):
                per_sl = (_BT // _L) * (d // _L)
                s_l = t3 // per_sl
                rem = t3 % per_sl
                b0 = (rem // (d // _L)) * _L
                dd0 = (rem % (d // _L)) * _L
                lrow = s_l * _BT + b0 + iota
                sl16 = jnp.full((_L,), s_l, jnp.int32)
                for kk in range(_L):
                    dcol = dd0 + ((iota + kk) & (_L - 1))
                    vals = plsc.load_gather(rs, [lrow, dcol])
                    sidx = dcol * _BT + b0 + iota
                    plsc.store_scatter(ts, [sl16, sidx], vals)

        def store_chunk(c, slot):
            bti = c // n_sg
            sg = c % n_sg
            for s_l in range(_SG):
                for dti in range(dt):
                    pltpu.async_copy(
                        trans[slot].at[s_l, pl.ds(dti * dr * _BT, dr * _BT)],
                        out_hbm.at[sg * _SG + s_l, dti, bt0 + bti],
                        ssem[slot])

        def run_chunk(c, slot):
            gather_wait(slot)

            @pl.when(c + 1 < n_chunks)
            def _():
                idx_wait(1 - slot)
                gather_start(1 - slot)

            @pl.when(c + 2 < n_chunks)
            def _():
                idx_start(c + 2, slot)

            @pl.when(c >= 2)
            def _():
                drain_stores(slot)

            transpose_chunk(slot)
            store_chunk(c, slot)

        idx_start(0, 0)
        idx_wait(0)
        gather_start(0)
        idx_start(1, 1)

        def body(t, carry):
            run_chunk(2 * t, 0)
            run_chunk(2 * t + 1, 1)
            return carry

        lax.fori_loop(0, n_chunks // 2, body, 0)
        drain_stores(0)
        drain_stores(1)

    return k


def kernel(x, table):
    b, s = x.shape
    v, d = table.shape
    xs = jnp.minimum(x.T.reshape(b * s), v - 1)
    out5 = _emb_call(b, s, d)(xs, table)
    out5 = out5.reshape(s, d // 8, b // 128, 8, 128)
    return out5.transpose(2, 4, 0, 1, 3).reshape(b, s, d)


# speedup vs baseline: 1.3845x; 1.0001x over previous
import functools

import jax
import jax.numpy as jnp
from jax import lax
from jax.experimental import pallas as pl
from jax.experimental.pallas import tpu as pltpu
from jax.experimental.pallas import tpu_sc as plsc

_NC = 2
_NS = 16
_NW = _NC * _NS
_L = 16

_SG = 5
_BT = 128


@functools.lru_cache(maxsize=None)
def _emb_call(b: int, s: int, d: int):
    per_w_b = b // _NW
    n_bt = per_w_b // _BT
    n_sg = s // _SG
    chunk = _SG * _BT
    n_chunks = n_bt * n_sg
    dt, dr = d // 8, 8

    mesh = plsc.VectorSubcoreMesh(core_axis_name="c", subcore_axis_name="s")

    @functools.partial(
        pl.kernel,
        mesh=mesh,
        out_type=jax.ShapeDtypeStruct((s, dt, b // _BT, dr * _BT), jnp.float32),
        compiler_params=pltpu.CompilerParams(
            use_tc_tiling_on_sc=False, needs_layout_passes=False),
        scratch_types=[
            pltpu.VMEM((chunk,), jnp.int32),
            pltpu.VMEM((chunk,), jnp.int32),
            pltpu.VMEM((chunk, d), jnp.float32),
            pltpu.VMEM((chunk, d), jnp.float32),
            pltpu.VMEM((_SG, d * _BT), jnp.float32),
            pltpu.VMEM((_SG, d * _BT), jnp.float32),
            pltpu.SemaphoreType.DMA,
            pltpu.SemaphoreType.DMA,
            pltpu.SemaphoreType.DMA,
            pltpu.SemaphoreType.DMA,
            pltpu.SemaphoreType.DMA,
            pltpu.SemaphoreType.DMA,
        ],
    )
    def k(table_hbm, xs_hbm, out_hbm, i0, i1, r0, r1, t0, t1,
          g0, g1, s0, s1, p0, p1):
        wid = lax.axis_index("s") * _NC + lax.axis_index("c")
        brow0 = wid * per_w_b
        bt0 = brow0 // _BT

        idx_c = (i0, i1)
        rows = (r0, r1)
        trans = (t0, t1)
        gsem = (g0, g1)
        ssem = (s0, s1)
        isem = (p0, p1)

        def idx_start(c, slot):
            bti = c // n_sg
            sg = c % n_sg
            for s_l in range(_SG):
                pltpu.async_copy(
                    xs_hbm.at[pl.ds((sg * _SG + s_l) * b + brow0 + bti * _BT,
                                    _BT)],
                    idx_c[slot].at[pl.ds(s_l * _BT, _BT)],
                    isem[slot])

        def idx_wait(slot):
            h = pltpu.make_async_copy(
                xs_hbm.at[pl.ds(0, _BT)],
                idx_c[slot].at[pl.ds(0, _BT)], isem[slot])
            for _ in range(_SG):
                h.wait()

        def gather_start(slot):
            pltpu.async_copy(table_hbm.at[idx_c[slot]], rows[slot],
                             gsem[slot])

        def gather_wait(slot):
            pltpu.make_async_copy(table_hbm.at[idx_c[slot]], rows[slot],
                                  gsem[slot]).wait()

        def drain_stores(slot):
            h = pltpu.make_async_copy(
                trans[slot].at[0, pl.ds(0, dr * _BT)], out_hbm.at[0, 0, 0],
                ssem[slot])
            for _ in range(_SG * dt):
                h.wait()

        def transpose_chunk(slot):
            rs = rows[slot]
            ts = trans[slot]
            iota = lax.iota(jnp.int32, _L)
            n_tiles = _SG * (_BT // _L) * (d // _L)

            @plsc.parallel_loop(0, n_tiles, unroll=2)
            def t_body(t3):
                per_sl = (_BT // _L) * (d // _L)
                s_l = t3 // per_sl
                rem = t3 % per_sl
                b0 = (rem // (d // _L)) * _L
                dd0 = (rem % (d // _L)) * _L
                lrow = s_l * _BT + b0 + iota
                sl16 = jnp.full((_L,), s_l, jnp.int32)
                for kk in range(_L):
                    dcol = dd0 + ((iota + kk) & (_L - 1))
                    vals = plsc.load_gather(rs, [lrow, dcol])
                    sidx = dcol * _BT + b0 + iota
                    plsc.store_scatter(ts, [sl16, sidx], vals)

        def store_chunk(c, slot):
            bti = c // n_sg
            sg = c % n_sg
            for s_l in range(_SG):
                for dti in range(dt):
                    pltpu.async_copy(
                        trans[slot].at[s_l, pl.ds(dti * dr * _BT, dr * _BT)],
                        out_hbm.at[sg * _SG + s_l, dti, bt0 + bti],
                        ssem[slot])

        def run_chunk(c, slot):
            gather_wait(slot)

            @pl.when(c + 1 < n_chunks)
            def _():
                idx_wait(1 - slot)
                gather_start(1 - slot)

            @pl.when(c + 2 < n_chunks)
            def _():
                idx_start(c + 2, slot)

            @pl.when(c >= 2)
            def _():
                drain_stores(slot)

            transpose_chunk(slot)
            store_chunk(c, slot)

        idx_start(0, 0)
        idx_wait(0)
        gather_start(0)
        idx_start(1, 1)

        def body(t, carry):
            run_chunk(2 * t, 0)
            run_chunk(2 * t + 1, 1)
            return carry

        lax.fori_loop(0, n_chunks // 2, body, 0)
        drain_stores(0)
        drain_stores(1)

    return k


def kernel(x, table):
    b, s = x.shape
    v, d = table.shape
    xs = jnp.minimum(x.T.reshape(b * s), v - 1)
    out5 = _emb_call(b, s, d)(table, xs)
    out5 = out5.reshape(s, d // 8, b // 128, 8, 128)
    return out5.transpose(2, 4, 0, 1, 3).reshape(b, s, d)


# speedup vs baseline: 1.4309x; 1.0335x over previous
import functools

import jax
import jax.numpy as jnp
from jax import lax
from jax.experimental import pallas as pl
from jax.experimental.pallas import tpu as pltpu
from jax.experimental.pallas import tpu_sc as plsc

_NC = 2
_NS = 16
_NW = _NC * _NS
_L = 16

_SG = 5
_BT = 128


@functools.lru_cache(maxsize=None)
def _emb_call(b: int, s: int, d: int):
    per_w_b = b // _NW
    n_bt = per_w_b // _BT
    n_sg = s // _SG
    chunk = _SG * _BT
    n_chunks = n_bt * n_sg
    dt, dr = d // 8, 8

    mesh = plsc.VectorSubcoreMesh(core_axis_name="c", subcore_axis_name="s")

    @functools.partial(
        pl.kernel,
        mesh=mesh,
        out_type=jax.ShapeDtypeStruct((s, dt, b // _BT, dr * _BT), jnp.float32),
        compiler_params=pltpu.CompilerParams(
            use_tc_tiling_on_sc=False, needs_layout_passes=False),
        scratch_types=[
            pltpu.VMEM((chunk,), jnp.int32),
            pltpu.VMEM((chunk,), jnp.int32),
            pltpu.VMEM((chunk, d), jnp.float32),
            pltpu.VMEM((chunk, d), jnp.float32),
            pltpu.VMEM((_SG, d * _BT), jnp.float32),
            pltpu.VMEM((_SG, d * _BT), jnp.float32),
            pltpu.SemaphoreType.DMA,
            pltpu.SemaphoreType.DMA,
            pltpu.SemaphoreType.DMA,
            pltpu.SemaphoreType.DMA,
            pltpu.SemaphoreType.DMA,
            pltpu.SemaphoreType.DMA,
        ],
    )
    def k(table_hbm, xs_hbm, out_hbm, i0, i1, r0, r1, t0, t1,
          g0, g1, s0, s1, p0, p1):
        wid = lax.axis_index("s") * _NC + lax.axis_index("c")
        brow0 = wid * per_w_b
        bt0 = brow0 // _BT

        idx_c = (i0, i1)
        rows = (r0, r1)
        trans = (t0, t1)
        gsem = (g0, g1)
        ssem = (s0, s1)
        isem = (p0, p1)

        def idx_start(c, slot):
            bti = c // n_sg
            sg = c % n_sg
            for s_l in range(_SG):
                pltpu.async_copy(
                    xs_hbm.at[pl.ds((sg * _SG + s_l) * b + brow0 + bti * _BT,
                                    _BT)],
                    idx_c[slot].at[pl.ds(s_l * _BT, _BT)],
                    isem[slot])

        def idx_wait(slot):
            h = pltpu.make_async_copy(
                xs_hbm.at[pl.ds(0, _BT)],
                idx_c[slot].at[pl.ds(0, _BT)], isem[slot])
            for _ in range(_SG):
                h.wait()

        def gather_start(slot):
            pltpu.async_copy(table_hbm.at[idx_c[slot]], rows[slot],
                             gsem[slot])

        def gather_wait(slot):
            pltpu.make_async_copy(table_hbm.at[idx_c[slot]], rows[slot],
                                  gsem[slot]).wait()

        def drain_stores(slot):
            h = pltpu.make_async_copy(
                trans[slot].at[0, pl.ds(0, dr * _BT)], out_hbm.at[0, 0, 0],
                ssem[slot])
            for _ in range(_SG * dt):
                h.wait()

        def transpose_chunk(slot):
            rs = rows[slot]
            ts = trans[slot]
            iota = lax.iota(jnp.int32, _L)
            n_tiles = _SG * (_BT // _L) * (d // _L)

            @plsc.parallel_loop(0, n_tiles, unroll=4)
            def t_body(t3):
                per_sl = (_BT // _L) * (d // _L)
                s_l = t3 // per_sl
                rem = t3 % per_sl
                b0 = (rem // (d // _L)) * _L
                dd0 = (rem % (d // _L)) * _L
                lrow = s_l * _BT + b0 + iota
                sl16 = jnp.full((_L,), s_l, jnp.int32)
                for kk in range(_L):
                    dcol = dd0 + ((iota + kk) & (_L - 1))
                    vals = plsc.load_gather(rs, [lrow, dcol])
                    sidx = dcol * _BT + b0 + iota
                    plsc.store_scatter(ts, [sl16, sidx], vals)

        def store_chunk(c, slot):
            bti = c // n_sg
            sg = c % n_sg
            for s_l in range(_SG):
                for dti in range(dt):
                    pltpu.async_copy(
                        trans[slot].at[s_l, pl.ds(dti * dr * _BT, dr * _BT)],
                        out_hbm.at[sg * _SG + s_l, dti, bt0 + bti],
                        ssem[slot])

        def run_chunk(c, slot):
            gather_wait(slot)

            @pl.when(c + 1 < n_chunks)
            def _():
                idx_wait(1 - slot)
                gather_start(1 - slot)

            @pl.when(c + 2 < n_chunks)
            def _():
                idx_start(c + 2, slot)

            @pl.when(c >= 2)
            def _():
                drain_stores(slot)

            transpose_chunk(slot)
            store_chunk(c, slot)

        idx_start(0, 0)
        idx_wait(0)
        gather_start(0)
        idx_start(1, 1)

        def body(t, carry):
            run_chunk(2 * t, 0)
            run_chunk(2 * t + 1, 1)
            return carry

        lax.fori_loop(0, n_chunks // 2, body, 0)
        drain_stores(0)
        drain_stores(1)

    return k


def kernel(x, table):
    b, s = x.shape
    v, d = table.shape
    xs = jnp.minimum(x.T.reshape(b * s), v - 1)
    out5 = _emb_call(b, s, d)(table, xs)
    out5 = out5.reshape(s, d // 8, b // 128, 8, 128)
    return out5.transpose(2, 4, 0, 1, 3).reshape(b, s, d)
